# baseline (speedup 1.0000x reference)
import jax
import jax.numpy as jnp
import numpy as np
from jax import lax
from jax.experimental import pallas as pl
from jax.experimental.pallas import tpu as pltpu
from jax.experimental.pallas import tpu_sc as plsc

N_POINTS = 262144
NUM_CORES = 2
NUM_SUBCORES = 16
NUM_TILES = NUM_CORES * NUM_SUBCORES
PER_TILE = N_POINTS // NUM_TILES
LANES = 16
VEC_ITERS = PER_TILE // LANES

SIZES = (np.float32(0.1), np.float32(0.2), np.float32(0.4))
GRIDS = (1024, 512, 256)
BASES = (0, 1024 * 1024, 1024 * 1024 + 512 * 512)
C = 1024 * 1024 + 512 * 512 + 256 * 256
SHIFT = np.float32(51.2)

CHUNK = 128
N_ROWS = 3 * PER_TILE // CHUNK
ROWS_PER_RES = N_ROWS // 3
ZB = 3072
ZERO_PER_SUBCORE = C // NUM_SUBCORES
N_ZERO_COPIES = ZERO_PER_SUBCORE // ZB


def _scatter_body(xs_hbm, ys_hbm, out_hbm, xv, yv, idxb, ones, zb, grid_sh,
                  sem_ld, sem_sc, sem_z):
    cid = lax.axis_index("c")
    sid = lax.axis_index("s")
    wid = cid * NUM_SUBCORES + sid

    base = wid * PER_TILE
    d_x = pltpu.async_copy(xs_hbm.at[pl.ds(base, PER_TILE)], xv, sem_ld)
    d_y = pltpu.async_copy(ys_hbm.at[pl.ds(base, PER_TILE)], yv, sem_ld)

    def _fill_zb(i, _):
        zb[pl.ds(i * LANES, LANES)] = jnp.zeros((LANES,), jnp.float32)
        return 0

    lax.fori_loop(0, ZB // LANES, _fill_zb, 0)

    def _fill_ones(i, _):
        ones[pl.ds(i * LANES, LANES)] = jnp.ones((LANES,), jnp.float32)
        return 0

    lax.fori_loop(0, CHUNK // LANES, _fill_ones, 0)

    zbase = sid * ZERO_PER_SUBCORE

    def _zero_fire(j, _):
        pltpu.async_copy(zb, grid_sh.at[pl.ds(zbase + j * ZB, ZB)], sem_z)
        return 0

    lax.fori_loop(0, N_ZERO_COPIES, _zero_fire, 0)

    d_x.wait()
    d_y.wait()

    def _quant(i, _):
        x = xv[pl.ds(i * LANES, LANES)]
        y = yv[pl.ds(i * LANES, LANES)]
        sx = x + SHIFT
        sy = y + SHIFT
        row = i // 8
        col = (i % 8) * LANES
        for r in range(3):
            g = GRIDS[r]
            cx = (sx / SIZES[r]).astype(jnp.int32)
            cy = (sy / SIZES[r]).astype(jnp.int32)
            cx = jnp.minimum(jnp.maximum(cx, 0), g - 1)
            cy = jnp.minimum(jnp.maximum(cy, 0), g - 1)
            idxb[r * ROWS_PER_RES + row, pl.ds(col, LANES)] = (
                cy * g + cx + BASES[r])
        return 0

    lax.fori_loop(0, VEC_ITERS, _quant, 0)

    def _zero_drain(j, _):
        pltpu.make_async_copy(
            zb, grid_sh.at[pl.ds(zbase + j * ZB, ZB)], sem_z).wait()
        return 0

    lax.fori_loop(0, N_ZERO_COPIES, _zero_drain, 0)
    plsc.subcore_barrier()

    def _scatter_fire(j, _):
        pltpu.async_copy(ones, grid_sh.at[idxb.at[j]], sem_sc)
        return 0

    lax.fori_loop(0, N_ROWS, _scatter_fire, 0)

    def _scatter_drain(j, _):
        pltpu.make_async_copy(ones, grid_sh.at[idxb.at[j]], sem_sc).wait()
        return 0

    lax.fori_loop(0, N_ROWS, _scatter_drain, 0)
    plsc.subcore_barrier()

    pltpu.sync_copy(grid_sh.at[pl.ds(zbase, ZERO_PER_SUBCORE)],
                    out_hbm.at[cid, pl.ds(zbase, ZERO_PER_SUBCORE)])


_scatter_call = pl.kernel(
    _scatter_body,
    out_type=jax.ShapeDtypeStruct((NUM_CORES, C), jnp.float32),
    mesh=plsc.VectorSubcoreMesh(core_axis_name="c", subcore_axis_name="s"),
    scratch_types=[
        pltpu.VMEM((PER_TILE,), jnp.float32),
        pltpu.VMEM((PER_TILE,), jnp.float32),
        pltpu.VMEM((N_ROWS, CHUNK), jnp.int32),
        pltpu.VMEM((CHUNK,), jnp.float32),
        pltpu.VMEM((ZB,), jnp.float32),
        pltpu.VMEM_SHARED((C,), jnp.float32),
        pltpu.SemaphoreType.DMA,
        pltpu.SemaphoreType.DMA,
        pltpu.SemaphoreType.DMA,
    ],
)

ROWS = C // 1024


def _reduce_body(a_ref, b_ref, c_ref, o0_ref, o1_ref, o2_ref):
    def cnt(ref):
        g = ref[...]
        occ = ((g[0] + g[1]) > 0.0).astype(jnp.float32)
        return jnp.sum(occ).astype(jnp.int32).reshape(1, 1, 1)

    o0_ref[...] = cnt(a_ref)
    o1_ref[...] = cnt(b_ref)
    o2_ref[...] = cnt(c_ref)


_reduce_call = pl.pallas_call(
    _reduce_body,
    grid=(32,),
    in_specs=[
        pl.BlockSpec((2, 32, 1024), lambda b: (0, b, 0)),
        pl.BlockSpec((2, 16, 1024), lambda b: (0, 64 + b % 16, 0)),
        pl.BlockSpec((2, 8, 1024), lambda b: (0, 160 + b % 8, 0)),
    ],
    out_specs=[
        pl.BlockSpec((1, 1, 1), lambda b: (b, 0, 0)),
        pl.BlockSpec((1, 1, 1), lambda b: (b % 16, 0, 0)),
        pl.BlockSpec((1, 1, 1), lambda b: (b % 8, 0, 0)),
    ],
    out_shape=[
        jax.ShapeDtypeStruct((32, 1, 1), jnp.int32),
        jax.ShapeDtypeStruct((16, 1, 1), jnp.int32),
        jax.ShapeDtypeStruct((8, 1, 1), jnp.int32),
    ],
)


def kernel(points_xy):
    ex = jnp.array([1.0, 0.0], dtype=jnp.float32)
    ey = jnp.array([0.0, 1.0], dtype=jnp.float32)
    xs = jnp.sum(points_xy * ex[None, :], axis=1)
    ys = jnp.sum(points_xy * ey[None, :], axis=1)
    grids = _scatter_call(xs, ys)
    g3 = grids.reshape(NUM_CORES, ROWS, 1024)
    o0, o1, o2 = _reduce_call(g3, g3, g3)
    return (o0.reshape(1, 32), o1.reshape(1, 16), o2.reshape(1, 8))


# speedup vs baseline: 1.3707x; 1.3707x over previous
import jax
import jax.numpy as jnp
import numpy as np
from jax import lax
from jax.experimental import pallas as pl
from jax.experimental.pallas import tpu as pltpu
from jax.experimental.pallas import tpu_sc as plsc

N_POINTS = 262144
NUM_CORES = 2
NUM_SUBCORES = 16
NUM_TILES = NUM_CORES * NUM_SUBCORES
PER_TILE = N_POINTS // NUM_TILES
LANES = 16
VEC_ITERS = PER_TILE // LANES

SIZES = (np.float32(0.1), np.float32(0.2), np.float32(0.4))
GRIDS = (1024, 512, 256)
BASES = (0, 1024 * 1024, 1024 * 1024 + 512 * 512)
C = 1024 * 1024 + 512 * 512 + 256 * 256
SHIFT = np.float32(51.2)

CHUNK = 128
N_ROWS = 3 * PER_TILE // CHUNK
ROWS_PER_RES = N_ROWS // 3
ZB = 3072
ZERO_PER_SUBCORE = C // NUM_SUBCORES
N_ZERO_COPIES = ZERO_PER_SUBCORE // ZB


def _scatter_body(xs_hbm, ys_hbm, out_hbm, xv, yv, idxb, ones, zb, grid_sh,
                  sem_ld, sem_sc, sem_z):
    cid = lax.axis_index("c")
    sid = lax.axis_index("s")
    wid = cid * NUM_SUBCORES + sid

    base = wid * PER_TILE
    d_x = pltpu.async_copy(xs_hbm.at[pl.ds(base, PER_TILE)], xv, sem_ld)
    d_y = pltpu.async_copy(ys_hbm.at[pl.ds(base, PER_TILE)], yv, sem_ld)

    def _fill_zb(i, _):
        zb[pl.ds(i * LANES, LANES)] = jnp.zeros((LANES,), jnp.float32)
        return 0

    lax.fori_loop(0, ZB // LANES, _fill_zb, 0)

    def _fill_ones(i, _):
        ones[pl.ds(i * LANES, LANES)] = jnp.ones((LANES,), jnp.float32)
        return 0

    lax.fori_loop(0, CHUNK // LANES, _fill_ones, 0)

    zbase = sid * ZERO_PER_SUBCORE

    def _zero_fire(j, _):
        pltpu.async_copy(zb, grid_sh.at[pl.ds(zbase + j * ZB, ZB)], sem_z)
        return 0

    lax.fori_loop(0, N_ZERO_COPIES, _zero_fire, 0)

    d_x.wait()
    d_y.wait()

    def _quant(i, _):
        x = xv[pl.ds(i * LANES, LANES)]
        y = yv[pl.ds(i * LANES, LANES)]
        sx = x + SHIFT
        sy = y + SHIFT
        row = i // 8
        col = (i % 8) * LANES
        for r in range(3):
            g = GRIDS[r]
            cx = (sx / SIZES[r]).astype(jnp.int32)
            cy = (sy / SIZES[r]).astype(jnp.int32)
            cx = jnp.minimum(jnp.maximum(cx, 0), g - 1)
            cy = jnp.minimum(jnp.maximum(cy, 0), g - 1)
            idxb[r * ROWS_PER_RES + row, pl.ds(col, LANES)] = (
                cy * g + cx + BASES[r])
        return 0

    lax.fori_loop(0, VEC_ITERS, _quant, 0)

    def _zero_drain(j, _):
        pltpu.make_async_copy(
            zb, grid_sh.at[pl.ds(zbase + j * ZB, ZB)], sem_z).wait()
        return 0

    lax.fori_loop(0, N_ZERO_COPIES, _zero_drain, 0)
    plsc.subcore_barrier()

    def _scatter_fire(j, _):
        pltpu.async_copy(ones, grid_sh.at[idxb.at[j]], sem_sc)
        return 0

    lax.fori_loop(0, N_ROWS, _scatter_fire, 0)

    def _scatter_drain(j, _):
        pltpu.make_async_copy(ones, grid_sh.at[idxb.at[j]], sem_sc).wait()
        return 0

    lax.fori_loop(0, N_ROWS, _scatter_drain, 0)
    plsc.subcore_barrier()

    pltpu.sync_copy(grid_sh.at[pl.ds(zbase, ZERO_PER_SUBCORE)],
                    out_hbm.at[pl.ds(cid * C + zbase, ZERO_PER_SUBCORE)])


_scatter_call = pl.kernel(
    _scatter_body,
    out_type=jax.ShapeDtypeStruct((NUM_CORES * C,), jnp.float32),
    mesh=plsc.VectorSubcoreMesh(core_axis_name="c", subcore_axis_name="s"),
    scratch_types=[
        pltpu.VMEM((PER_TILE,), jnp.float32),
        pltpu.VMEM((PER_TILE,), jnp.float32),
        pltpu.VMEM((N_ROWS, CHUNK), jnp.int32),
        pltpu.VMEM((CHUNK,), jnp.float32),
        pltpu.VMEM((ZB,), jnp.float32),
        pltpu.VMEM_SHARED((C,), jnp.float32),
        pltpu.SemaphoreType.DMA,
        pltpu.SemaphoreType.DMA,
        pltpu.SemaphoreType.DMA,
    ],
)

S0 = 32 * 1024
S1 = 32 * 512
S2 = 32 * 256


def _reduce_body(a0, a1, b0, b1, c0, c1, o0_ref, o1_ref, o2_ref):
    b = pl.program_id(0)

    def cnt(r0, r1):
        occ = ((r0[...] + r1[...]) > 0.0).astype(jnp.float32)
        return jnp.sum(occ).astype(jnp.int32)

    def put(ref, lane, val):
        li = lax.broadcasted_iota(jnp.int32, ref.shape, 2)
        ref[...] = jnp.where(li == lane, val, ref[...])

    put(o0_ref, b, cnt(a0, a1))
    put(o1_ref, b % 16, cnt(b0, b1))
    put(o2_ref, b % 8, cnt(c0, c1))


_reduce_call = pl.pallas_call(
    _reduce_body,
    grid=(32,),
    in_specs=[
        pl.BlockSpec((S0,), lambda b: (b,)),
        pl.BlockSpec((S0,), lambda b: (C // S0 + b,)),
        pl.BlockSpec((S1,), lambda b: (BASES[1] // S1 + b % 16,)),
        pl.BlockSpec((S1,), lambda b: ((C + BASES[1]) // S1 + b % 16,)),
        pl.BlockSpec((S2,), lambda b: (BASES[2] // S2 + b % 8,)),
        pl.BlockSpec((S2,), lambda b: ((C + BASES[2]) // S2 + b % 8,)),
    ],
    out_specs=[
        pl.BlockSpec((1, 1, 32), lambda b: (0, 0, 0)),
        pl.BlockSpec((1, 1, 16), lambda b: (0, 0, 0)),
        pl.BlockSpec((1, 1, 8), lambda b: (0, 0, 0)),
    ],
    out_shape=[
        jax.ShapeDtypeStruct((1, 1, 32), jnp.int32),
        jax.ShapeDtypeStruct((1, 1, 16), jnp.int32),
        jax.ShapeDtypeStruct((1, 1, 8), jnp.int32),
    ],
)


def kernel(points_xy):
    grids = _scatter_call(points_xy[:, 0], points_xy[:, 1])
    o0, o1, o2 = _reduce_call(grids, grids, grids, grids, grids, grids)
    return (o0.reshape(1, 32), o1.reshape(1, 16), o2.reshape(1, 8))


# speedup vs baseline: 1.4322x; 1.0449x over previous
import jax
import jax.numpy as jnp
import numpy as np
from jax import lax
from jax.experimental import pallas as pl
from jax.experimental.pallas import tpu as pltpu
from jax.experimental.pallas import tpu_sc as plsc

N_POINTS = 262144
NUM_CORES = 2
NUM_SUBCORES = 16
NUM_TILES = NUM_CORES * NUM_SUBCORES
PER_TILE = N_POINTS // NUM_TILES
LANES = 16
VEC_ITERS = PER_TILE // LANES

SIZES = (np.float32(0.1), np.float32(0.2), np.float32(0.4))
GRIDS = (1024, 512, 256)
BASES = (0, 1024 * 1024, 1024 * 1024 + 512 * 512)
C = 1024 * 1024 + 512 * 512 + 256 * 256
SHIFT = np.float32(51.2)

CHUNK = 128
N_ROWS = 3 * PER_TILE // CHUNK
ROWS_PER_RES = N_ROWS // 3
ZB = 3072
ZERO_PER_SUBCORE = C // NUM_SUBCORES
N_ZERO_COPIES = ZERO_PER_SUBCORE // ZB


def _scatter_body(xs_hbm, ys_hbm, out_hbm, xv, yv, idxb, ones, zb, grid_sh,
                  sem_ld, sem_sc, sem_z):
    cid = lax.axis_index("c")
    sid = lax.axis_index("s")
    wid = cid * NUM_SUBCORES + sid

    base = wid * PER_TILE
    d_x = pltpu.async_copy(xs_hbm.at[pl.ds(base, PER_TILE)], xv, sem_ld)
    d_y = pltpu.async_copy(ys_hbm.at[pl.ds(base, PER_TILE)], yv, sem_ld)

    def _fill_zb(i, _):
        zb[pl.ds(i * LANES, LANES)] = jnp.zeros((LANES,), jnp.float32)
        return 0

    lax.fori_loop(0, ZB // LANES, _fill_zb, 0)

    def _fill_ones(i, _):
        ones[pl.ds(i * LANES, LANES)] = jnp.ones((LANES,), jnp.float32)
        return 0

    lax.fori_loop(0, CHUNK // LANES, _fill_ones, 0)

    zbase = sid * ZERO_PER_SUBCORE

    def _zero_fire(j, _):
        pltpu.async_copy(zb, grid_sh.at[pl.ds(zbase + j * ZB, ZB)], sem_z)
        return 0

    lax.fori_loop(0, N_ZERO_COPIES, _zero_fire, 0)

    d_x.wait()
    d_y.wait()

    def _quant(i, _):
        x = xv[pl.ds(i * LANES, LANES)]
        y = yv[pl.ds(i * LANES, LANES)]
        sx = x + SHIFT
        sy = y + SHIFT
        row = i // 8
        col = (i % 8) * LANES
        for r in range(3):
            g = GRIDS[r]
            cx = (sx / SIZES[r]).astype(jnp.int32)
            cy = (sy / SIZES[r]).astype(jnp.int32)
            cx = jnp.minimum(jnp.maximum(cx, 0), g - 1)
            cy = jnp.minimum(jnp.maximum(cy, 0), g - 1)
            idxb[r * ROWS_PER_RES + row, pl.ds(col, LANES)] = (
                cy * g + cx + BASES[r])
        return 0

    lax.fori_loop(0, VEC_ITERS, _quant, 0)

    def _zero_drain(j, _):
        pltpu.make_async_copy(
            zb, grid_sh.at[pl.ds(zbase + j * ZB, ZB)], sem_z).wait()
        return 0

    lax.fori_loop(0, N_ZERO_COPIES, _zero_drain, 0)
    plsc.subcore_barrier()

    def _scatter_fire(j, _):
        pltpu.async_copy(ones, grid_sh.at[idxb.at[j]], sem_sc)
        return 0

    lax.fori_loop(0, N_ROWS, _scatter_fire, 0)

    def _scatter_drain(j, _):
        pltpu.make_async_copy(ones, grid_sh.at[idxb.at[j]], sem_sc).wait()
        return 0

    lax.fori_loop(0, N_ROWS, _scatter_drain, 0)
    plsc.subcore_barrier()

    pltpu.sync_copy(grid_sh.at[pl.ds(zbase, ZERO_PER_SUBCORE)],
                    out_hbm.at[pl.ds(cid * C + zbase, ZERO_PER_SUBCORE)])


_scatter_call = pl.kernel(
    _scatter_body,
    out_type=jax.ShapeDtypeStruct((NUM_CORES * C,), jnp.float32),
    mesh=plsc.VectorSubcoreMesh(core_axis_name="c", subcore_axis_name="s"),
    scratch_types=[
        pltpu.VMEM((PER_TILE,), jnp.float32),
        pltpu.VMEM((PER_TILE,), jnp.float32),
        pltpu.VMEM((N_ROWS, CHUNK), jnp.int32),
        pltpu.VMEM((CHUNK,), jnp.float32),
        pltpu.VMEM((ZB,), jnp.float32),
        pltpu.VMEM_SHARED((C,), jnp.float32),
        pltpu.SemaphoreType.DMA,
        pltpu.SemaphoreType.DMA,
        pltpu.SemaphoreType.DMA,
    ],
)

S0 = 32 * 1024
S1 = 32 * 512
S2 = 32 * 256


def _reduce_body(a0, a1, b0, b1, c0, c1, o0_ref, o1_ref, o2_ref):
    b = pl.program_id(0)

    def cnt(r0, r1):
        n = r0.shape[0]
        a = r0[...].reshape(n // 1024, 1024)
        b2 = r1[...].reshape(n // 1024, 1024)
        occ = ((a + b2) > 0.0).astype(jnp.float32)
        return jnp.sum(occ).astype(jnp.int32)

    def put(ref, lane, val):
        li = lax.broadcasted_iota(jnp.int32, ref.shape, 2)
        ref[...] = jnp.where(li == lane, val, ref[...])

    put(o0_ref, b, cnt(a0, a1))
    put(o1_ref, b % 16, cnt(b0, b1))
    put(o2_ref, b % 8, cnt(c0, c1))


_reduce_call = pl.pallas_call(
    _reduce_body,
    grid=(32,),
    in_specs=[
        pl.BlockSpec((S0,), lambda b: (b,)),
        pl.BlockSpec((S0,), lambda b: (C // S0 + b,)),
        pl.BlockSpec((S1,), lambda b: (BASES[1] // S1 + b % 16,)),
        pl.BlockSpec((S1,), lambda b: ((C + BASES[1]) // S1 + b % 16,)),
        pl.BlockSpec((S2,), lambda b: (BASES[2] // S2 + b % 8,)),
        pl.BlockSpec((S2,), lambda b: ((C + BASES[2]) // S2 + b % 8,)),
    ],
    out_specs=[
        pl.BlockSpec((1, 1, 32), lambda b: (0, 0, 0)),
        pl.BlockSpec((1, 1, 16), lambda b: (0, 0, 0)),
        pl.BlockSpec((1, 1, 8), lambda b: (0, 0, 0)),
    ],
    out_shape=[
        jax.ShapeDtypeStruct((1, 1, 32), jnp.int32),
        jax.ShapeDtypeStruct((1, 1, 16), jnp.int32),
        jax.ShapeDtypeStruct((1, 1, 8), jnp.int32),
    ],
)


def kernel(points_xy):
    grids = _scatter_call(points_xy[:, 0], points_xy[:, 1])
    o0, o1, o2 = _reduce_call(grids, grids, grids, grids, grids, grids)
    return (o0.reshape(1, 32), o1.reshape(1, 16), o2.reshape(1, 8))


# speedup vs baseline: 1.4641x; 1.0223x over previous
import jax
import jax.numpy as jnp
import numpy as np
from jax import lax
from jax.experimental import pallas as pl
from jax.experimental.pallas import tpu as pltpu
from jax.experimental.pallas import tpu_sc as plsc

N_POINTS = 262144
NUM_CORES = 2
NUM_SUBCORES = 16
NUM_TILES = NUM_CORES * NUM_SUBCORES
PER_TILE = N_POINTS // NUM_TILES
LANES = 16
VEC_ITERS = PER_TILE // LANES

SIZES = (np.float32(0.1), np.float32(0.2), np.float32(0.4))
GRIDS = (1024, 512, 256)
BASES = (0, 1024 * 1024, 1024 * 1024 + 512 * 512)
C = 1024 * 1024 + 512 * 512 + 256 * 256
SHIFT = np.float32(51.2)

CHUNK = 128
N_ROWS = 3 * PER_TILE // CHUNK
ROWS_PER_RES = N_ROWS // 3
ROWS_PER_STREAM = 8
N_STREAMS = N_ROWS // ROWS_PER_STREAM
ZB = 2048
ZERO_PER_SUBCORE = C // NUM_SUBCORES
N_ZERO_COPIES = ZERO_PER_SUBCORE // ZB


def _scatter_body(xs_hbm, ys_hbm, out_hbm, xv, yv, idxb, ones, zb, grid_sh,
                  sem_ld, sem_sc, sem_z):
    cid = lax.axis_index("c")
    sid = lax.axis_index("s")
    wid = cid * NUM_SUBCORES + sid

    base = wid * PER_TILE
    d_x = pltpu.async_copy(xs_hbm.at[pl.ds(base, PER_TILE)], xv, sem_ld)
    d_y = pltpu.async_copy(ys_hbm.at[pl.ds(base, PER_TILE)], yv, sem_ld)

    def _fill_zb(i, _):
        zb[pl.ds(i * LANES, LANES)] = jnp.zeros((LANES,), jnp.float32)
        return 0

    lax.fori_loop(0, ZB // LANES, _fill_zb, 0)

    def _fill_ones(i, _):
        ones[i // 8, pl.ds((i % 8) * LANES, LANES)] = jnp.ones(
            (LANES,), jnp.float32)
        return 0

    lax.fori_loop(0, ROWS_PER_STREAM * CHUNK // LANES, _fill_ones, 0)
    ones_r = ones.at[0]

    zbase = sid * ZERO_PER_SUBCORE

    def _zero_fire(j, _):
        pltpu.async_copy(zb, grid_sh.at[pl.ds(zbase + j * ZB, ZB)], sem_z)
        return 0

    lax.fori_loop(0, N_ZERO_COPIES, _zero_fire, 0)

    d_x.wait()
    d_y.wait()

    def _quant(i, _):
        x = xv[pl.ds(i * LANES, LANES)]
        y = yv[pl.ds(i * LANES, LANES)]
        sx = x + SHIFT
        sy = y + SHIFT
        row = i // 8
        col = (i % 8) * LANES
        cx = (sx / SIZES[0]).astype(jnp.int32)
        cy = (sy / SIZES[0]).astype(jnp.int32)
        cx = jnp.minimum(jnp.maximum(cx, 0), GRIDS[0] - 1)
        cy = jnp.minimum(jnp.maximum(cy, 0), GRIDS[0] - 1)
        idxb[row, pl.ds(col, LANES)] = cy * 1024 + cx
        idxb[ROWS_PER_RES + row, pl.ds(col, LANES)] = (
            (cy >> 1) * 512 + (cx >> 1) + BASES[1])
        idxb[2 * ROWS_PER_RES + row, pl.ds(col, LANES)] = (
            (cy >> 2) * 256 + (cx >> 2) + BASES[2])
        return 0

    lax.fori_loop(0, VEC_ITERS, _quant, 0)

    def _zero_drain(j, _):
        pltpu.make_async_copy(
            zb, grid_sh.at[pl.ds(zbase + j * ZB, ZB)], sem_z).wait()
        return 0

    lax.fori_loop(0, N_ZERO_COPIES, _zero_drain, 0)
    plsc.subcore_barrier()

    def _scatter_fire(j, _):
        pltpu.async_copy(ones_r, grid_sh.at[idxb.at[j]], sem_sc)
        return 0

    lax.fori_loop(0, N_ROWS, _scatter_fire, 0)

    def _scatter_drain(j, _):
        pltpu.make_async_copy(ones_r, grid_sh.at[idxb.at[j]], sem_sc).wait()
        return 0

    lax.fori_loop(0, N_ROWS, _scatter_drain, 0)
    plsc.subcore_barrier()

    pltpu.sync_copy(grid_sh.at[pl.ds(zbase, ZERO_PER_SUBCORE)],
                    out_hbm.at[pl.ds(cid * C + zbase, ZERO_PER_SUBCORE)])


_scatter_call = pl.kernel(
    _scatter_body,
    out_type=jax.ShapeDtypeStruct((NUM_CORES * C,), jnp.float32),
    mesh=plsc.VectorSubcoreMesh(core_axis_name="c", subcore_axis_name="s"),
    scratch_types=[
        pltpu.VMEM((PER_TILE,), jnp.float32),
        pltpu.VMEM((PER_TILE,), jnp.float32),
        pltpu.VMEM((N_ROWS, CHUNK), jnp.int32),
        pltpu.VMEM((ROWS_PER_STREAM, CHUNK), jnp.float32),
        pltpu.VMEM((ZB,), jnp.float32),
        pltpu.VMEM_SHARED((C,), jnp.float32),
        pltpu.SemaphoreType.DMA,
        pltpu.SemaphoreType.DMA,
        pltpu.SemaphoreType.DMA,
    ],
)

S0 = 32 * 1024
S1 = 32 * 512
S2 = 32 * 256


def _reduce_body(a0, a1, b0, b1, c0, c1, o0_ref, o1_ref, o2_ref):
    b = pl.program_id(0)

    def cnt(r0, r1):
        n = r0.shape[0]
        a = r0[...].reshape(n // 1024, 1024)
        b2 = r1[...].reshape(n // 1024, 1024)
        occ = ((a + b2) > 0.0).astype(jnp.float32)
        return jnp.sum(occ).astype(jnp.int32)

    def put(ref, lane, val):
        li = lax.broadcasted_iota(jnp.int32, ref.shape, 2)
        ref[...] = jnp.where(li == lane, val, ref[...])

    put(o0_ref, b, cnt(a0, a1))
    put(o1_ref, b % 16, cnt(b0, b1))
    put(o2_ref, b % 8, cnt(c0, c1))


_reduce_call = pl.pallas_call(
    _reduce_body,
    grid=(32,),
    in_specs=[
        pl.BlockSpec((S0,), lambda b: (b,)),
        pl.BlockSpec((S0,), lambda b: (C // S0 + b,)),
        pl.BlockSpec((S1,), lambda b: (BASES[1] // S1 + b % 16,)),
        pl.BlockSpec((S1,), lambda b: ((C + BASES[1]) // S1 + b % 16,)),
        pl.BlockSpec((S2,), lambda b: (BASES[2] // S2 + b % 8,)),
        pl.BlockSpec((S2,), lambda b: ((C + BASES[2]) // S2 + b % 8,)),
    ],
    out_specs=[
        pl.BlockSpec((1, 1, 32), lambda b: (0, 0, 0)),
        pl.BlockSpec((1, 1, 16), lambda b: (0, 0, 0)),
        pl.BlockSpec((1, 1, 8), lambda b: (0, 0, 0)),
    ],
    out_shape=[
        jax.ShapeDtypeStruct((1, 1, 32), jnp.int32),
        jax.ShapeDtypeStruct((1, 1, 16), jnp.int32),
        jax.ShapeDtypeStruct((1, 1, 8), jnp.int32),
    ],
)


def kernel(points_xy):
    grids = _scatter_call(points_xy[:, 0], points_xy[:, 1])
    o0, o1, o2 = _reduce_call(grids, grids, grids, grids, grids, grids)
    return (o0.reshape(1, 32), o1.reshape(1, 16), o2.reshape(1, 8))
